# baseline (speedup 1.0000x reference)
import jax
import jax.numpy as jnp
from jax.experimental import pallas as pl

_BB = 512


def _body(x_ref, m_ref, o_ref):
    d = x_ref.shape[2]
    zero = jnp.zeros((), x_ref.dtype)
    o_ref[:, :d] = jnp.where(m_ref[:, 0:1], x_ref[:, 0, :], zero)
    o_ref[:, d:] = jnp.where(m_ref[:, 1:2], x_ref[:, 1, :], zero)


def kernel(x, head_tail_mask):
    b, s, d = x.shape
    return pl.pallas_call(
        _body,
        grid=(b // _BB,),
        in_specs=[
            pl.BlockSpec((_BB, s, d), lambda i: (i, 0, 0)),
            pl.BlockSpec((_BB, s), lambda i: (i, 0)),
        ],
        out_specs=pl.BlockSpec((_BB, s * d), lambda i: (i, 0)),
        out_shape=jax.ShapeDtypeStruct((b, s * d), x.dtype),
    )(x, head_tail_mask)


# speedup vs baseline: 1.0194x; 1.0194x over previous
import jax
import jax.numpy as jnp
from jax.experimental import pallas as pl

_BB = 1024


def _body(x_ref, m_ref, o_ref):
    d = x_ref.shape[2]
    zero = jnp.zeros((), x_ref.dtype)
    o_ref[:, :d] = jnp.where(m_ref[:, 0:1], x_ref[:, 0, :], zero)
    o_ref[:, d:] = jnp.where(m_ref[:, 1:2], x_ref[:, 1, :], zero)


def kernel(x, head_tail_mask):
    b, s, d = x.shape
    return pl.pallas_call(
        _body,
        grid=(b // _BB,),
        in_specs=[
            pl.BlockSpec((_BB, s, d), lambda i: (i, 0, 0)),
            pl.BlockSpec((_BB, s), lambda i: (i, 0)),
        ],
        out_specs=pl.BlockSpec((_BB, s * d), lambda i: (i, 0)),
        out_shape=jax.ShapeDtypeStruct((b, s * d), x.dtype),
    )(x, head_tail_mask)
